# baseline (speedup 1.0000x reference)
import functools
import jax
import jax.numpy as jnp
from jax import lax
from jax.experimental import pallas as pl
from jax.experimental.pallas import tpu as pltpu
from jax.experimental.pallas import tpu_sc as plsc

_N = 10000
_D = 128
_H = 128
_E = 320000
_B = 8

_NC = 2
_NS = 16
_NW = _NC * _NS
_K = 64
_EPW = 20480
_EP = _NW * _EPW
_ER = _E // 128
_PADR = _EP // 128 - 2 * _ER
_CPT = _EPW // _K
_C0_CHUNKS = 152
_C1_CHUNKS = 2 * _CPT - _C0_CHUNKS
_NR0 = _C0_CHUNKS // 2
_NR1 = _C1_CHUNKS // 2
_ROWS_PER_TILE = 624


def _idx_body(us_ref, uo_ref, ud_ref, ds_ref, do_ref, dd_ref, idx2_ref):
    su = us_ref[...] * 4 + (uo_ref[...] < 0).astype(jnp.int32)
    sd = ds_ref[...] * 4 + 2 + (do_ref[...] < 0).astype(jnp.int32)
    src_all = jnp.concatenate([su, sd, jnp.zeros((_PADR, 128), jnp.int32)],
                              axis=0)
    dst_all = jnp.concatenate([ud_ref[...], dd_ref[...],
                               jnp.full((_PADR, 128), _N, jnp.int32)], axis=0)
    idx2_ref[...] = jnp.stack(
        [src_all[:, 0:64], dst_all[:, 0:64],
         src_all[:, 64:128], dst_all[:, 64:128]], axis=1)


def _build_indices(up_src, up_orient, up_dst, down_src, down_orient, down_dst):
    out = pl.pallas_call(
        _idx_body,
        out_shape=jax.ShapeDtypeStruct((_EP // 128, 4, 64), jnp.int32),
    )(up_src.reshape(_ER, 128), up_orient.reshape(_ER, 128),
      up_dst.reshape(_ER, 128), down_src.reshape(_ER, 128),
      down_orient.reshape(_ER, 128), down_dst.reshape(_ER, 128))
    return out.reshape(_EP // 32, 64)


_RB = 2000


def _mm_body1(x_ref, wcat_ref, wt_ref, t_ref, hw_ref):
    xs = x_ref[...]
    t_ref[...] = jnp.dot(xs, wcat_ref[...], preferred_element_type=jnp.float32)
    hw_ref[...] = jnp.dot(xs, wt_ref[...], preferred_element_type=jnp.float32)


def _mm_body3(a0_ref, a1_ref, hwp_ref, wcat_ref, wt_ref, t_ref, hw_ref):
    xs = a0_ref[...] + a1_ref[...] + hwp_ref[...]
    t_ref[...] = jnp.dot(xs, wcat_ref[...], preferred_element_type=jnp.float32)
    hw_ref[...] = jnp.dot(xs, wt_ref[...], preferred_element_type=jnp.float32)


def _layer_matmul(terms, wcat_t, w_t):
    body = _mm_body1 if len(terms) == 1 else _mm_body3
    row_spec = pl.BlockSpec((_RB, 128), lambda i: (i, 0))
    in_specs = [row_spec] * len(terms) + [
        pl.BlockSpec((128, 512), lambda i: (0, 0)),
        pl.BlockSpec((128, 128), lambda i: (0, 0)),
    ]
    t_raw, hw = pl.pallas_call(
        body,
        grid=(_N // _RB,),
        in_specs=in_specs,
        out_specs=(
            pl.BlockSpec((_RB, 512), lambda i: (i, 0)),
            pl.BlockSpec((_RB, 128), lambda i: (i, 0)),
        ),
        out_shape=(
            jax.ShapeDtypeStruct((_N, 512), jnp.float32),
            jax.ShapeDtypeStruct((_N, 128), jnp.float32),
        ),
    )(*terms, wcat_t, w_t)
    return t_raw.reshape(4 * _N, 128), hw


_SC_MESH = plsc.VectorSubcoreMesh(core_axis_name="c", subcore_axis_name="s")


@functools.partial(
    pl.kernel,
    out_type=jax.ShapeDtypeStruct((_NC, _N, 128), jnp.float32),
    mesh=_SC_MESH,
    scratch_types=(
        [pltpu.VMEM((8, 64), jnp.int32)] * 2 +
        [pltpu.VMEM((_K, 128), jnp.float32)] * 4 +
        [pltpu.VMEM_SHARED((_N + 16, 128), jnp.float32)] +
        [pltpu.SemaphoreType.DMA] * 2 +
        [pltpu.SemaphoreType.DMA] * 4 +
        [pltpu.SemaphoreType.DMA] * 4
    ),
)
def _sc_scatter(table_hbm, idx2_hbm, zeros_hbm, out_hbm, *scr):
    ibuf = scr[0:2]
    rows = (scr[2:4], scr[4:6])
    acc = scr[6]
    isem = scr[7:9]
    gsem = (scr[9:11], scr[11:13])
    ssem = (scr[13:15], scr[15:17])
    c = lax.axis_index("c")
    s = lax.axis_index("s")
    chunk_base = lax.select(c == 0, s * _C0_CHUNKS,
                            _NS * _C0_CHUNKS + s * _C1_CHUNKS)
    my_nr = lax.select(c == 0, _NR0, _NR1)
    ibase = 2 * chunk_base
    r0 = s * _ROWS_PER_TILE
    tail = _NS * _ROWS_PER_TILE
    pltpu.sync_copy(zeros_hbm.at[pl.ds(r0, _ROWS_PER_TILE)],
                    acc.at[pl.ds(r0, _ROWS_PER_TILE)])

    @pl.when(s == _NS - 1)
    def _():
        pltpu.sync_copy(zeros_hbm.at[pl.ds(tail, _N - tail)],
                        acc.at[pl.ds(tail, _N - tail)])

    plsc.subcore_barrier()

    def ifetch(p, m):
        pltpu.async_copy(idx2_hbm.at[pl.ds(ibase + 8 * p, 8)],
                         ibuf[m], isem[m])

    def iwait(p, m):
        pltpu.make_async_copy(idx2_hbm.at[pl.ds(ibase + 8 * p, 8)],
                              ibuf[m], isem[m]).wait()

    def start_gathers(m, h, g):
        for b in range(2):
            pltpu.async_copy(table_hbm.at[ibuf[m].at[4 * h + 2 * b]],
                             rows[g][b], gsem[g][b])

    def wait_gather(m, h, g, b):
        pltpu.make_async_copy(table_hbm.at[ibuf[m].at[4 * h + 2 * b]],
                              rows[g][b], gsem[g][b]).wait()

    def start_scatter(m, h, g, b):
        pltpu.async_copy(rows[g][b], acc.at[ibuf[m].at[4 * h + 2 * b + 1]],
                         ssem[g][b], add=True)

    def wait_scatter(m, h, g, b):
        pltpu.make_async_copy(rows[g][b], acc.at[ibuf[m].at[4 * h + 2 * b + 1]],
                              ssem[g][b]).wait()

    ifetch(0, 0)
    iwait(0, 0)
    start_gathers(0, 0, 0)

    def round_t(t, u):
        g = u % 2
        m, h = (u // 2) % 2, u % 2
        un = (u + 1) % 4
        mn, hn = (un // 2) % 2, un % 2
        up = (u + 3) % 4
        mp, hp = (up // 2) % 2, up % 2
        for b in range(2):
            wait_gather(m, h, g, b)
            start_scatter(m, h, g, b)

        @pl.when(t + 1 < my_nr)
        def _():
            @pl.when(t >= 1)
            def _():
                for b in range(2):
                    wait_scatter(mp, hp, 1 - g, b)
            if un % 2 == 0:
                iwait((t + 1) // 2, mn)
            start_gathers(mn, hn, 1 - g)

        if u % 2 == 0:
            @pl.when(t + 2 < my_nr)
            def _():
                ifetch((t + 2) // 2, ((u + 2) // 2) % 2)

    def quad_body(qq, carry):
        for u in range(4):
            round_t(4 * qq + u, u)
        return carry

    lax.fori_loop(0, my_nr // 4, quad_body, 0)
    for b in range(2):
        wait_scatter(1, 0, 0, b)
    for b in range(2):
        wait_scatter(1, 1, 1, b)
    plsc.subcore_barrier()
    pltpu.sync_copy(acc.at[pl.ds(r0, _ROWS_PER_TILE)],
                    out_hbm.at[c, pl.ds(r0, _ROWS_PER_TILE)])

    @pl.when(s == _NS - 1)
    def _():
        pltpu.sync_copy(acc.at[pl.ds(tail, _N - tail)],
                        out_hbm.at[c, pl.ds(tail, _N - tail)])


def _pool_body(a0_ref, a1_ref, hw_ref, bt_ref, w1_ref, b1_ref, w2_ref, b2_ref,
               out_ref, pooled_ref):
    i = pl.program_id(0)
    h = jnp.abs(a0_ref[...] + a1_ref[...] + hw_ref[...])
    onehot = (bt_ref[...] == lax.broadcasted_iota(jnp.int32, (1, _B), 1)
              ).astype(jnp.float32)
    part = lax.dot_general(onehot, h, (((0,), (0,)), ((), ())),
                           preferred_element_type=jnp.float32)

    @pl.when(i == 0)
    def _():
        pooled_ref[...] = part

    @pl.when(i > 0)
    def _():
        pooled_ref[...] += part

    @pl.when(i == _N // _RB - 1)
    def _():
        p = pooled_ref[...]
        h1 = jnp.maximum(
            jnp.dot(p, w1_ref[...], preferred_element_type=jnp.float32)
            + b1_ref[...], 0.0)
        out_ref[...] = jnp.dot(h1, w2_ref[...],
                               preferred_element_type=jnp.float32) + b2_ref[...]


def _pool_mlp(a0, a1, hw, batch2d, w1t, b1, w2t, b2):
    row_spec = pl.BlockSpec((_RB, 128), lambda i: (i, 0))
    const = lambda shape: pl.BlockSpec(shape, lambda i: (0, 0))
    return pl.pallas_call(
        _pool_body,
        grid=(_N // _RB,),
        in_specs=[row_spec, row_spec, row_spec,
                  pl.BlockSpec((_RB, 1), lambda i: (i, 0)),
                  const((128, 128)), const((1, 128)),
                  const((128, 128)), const((1, 128))],
        out_specs=const((_B, 128)),
        out_shape=jax.ShapeDtypeStruct((_B, 128), jnp.float32),
        scratch_shapes=[pltpu.VMEM((_B, 128), jnp.float32)],
    )(a0, a1, hw, batch2d, w1t, b1, w2t, b2)


def kernel(x, up_index, up_orient, down_index, down_orient, batch,
           W_up_0, W_down_0, W_0, W_up_1, W_down_1, W_1, W_up_2, W_down_2, W_2,
           lin1_W, lin1_b, lin2_W, lin2_b):
    f32 = jnp.float32
    idx2 = _build_indices(up_index[0], up_orient, up_index[1],
                          down_index[0], down_orient, down_index[1])
    zeros = jnp.zeros((_N, 128), f32)

    def wcat(Wu, Wd):
        return jnp.concatenate([Wu.T, -Wu.T, Wd.T, -Wd.T], axis=1)

    layers = ((W_up_0, W_down_0, W_0), (W_up_1, W_down_1, W_1),
              (W_up_2, W_down_2, W_2))

    terms = (x,)
    for Wu, Wd, W in layers:
        t_tab, hw = _layer_matmul(terms, wcat(Wu, Wd), W.T)
        acc = _sc_scatter(t_tab, idx2, zeros)
        terms = (acc[0], acc[1], hw)

    w2p = jnp.zeros((128, 128), f32).at[:, :2].set(lin2_W.T)
    b2p = jnp.zeros((1, 128), f32).at[0, :2].set(lin2_b)
    out = _pool_mlp(terms[0], terms[1], terms[2], batch.reshape(_N, 1),
                    lin1_W.T, lin1_b.reshape(1, 128), w2p, b2p)
    return out[:, :2]


# speedup vs baseline: 1.0740x; 1.0740x over previous
import functools
import jax
import jax.numpy as jnp
from jax import lax
from jax.experimental import pallas as pl
from jax.experimental.pallas import tpu as pltpu
from jax.experimental.pallas import tpu_sc as plsc

_N = 10000
_D = 128
_H = 128
_E = 320000
_B = 8

_NC = 2
_NS = 16
_NW = _NC * _NS
_K = 64
_EPW = 20480
_EP = _NW * _EPW
_ER = _E // 128
_PADR = _EP // 128 - 2 * _ER
_CPT = _EPW // _K
_C0_CHUNKS = _CPT
_C1_CHUNKS = 2 * _CPT - _C0_CHUNKS
_NR0 = _C0_CHUNKS // 2
_NR1 = _C1_CHUNKS // 2
_ROWS_PER_TILE = 624


def _idx_body(us_ref, uo_ref, ud_ref, ds_ref, do_ref, dd_ref, idx2_ref):
    su = us_ref[...] * 4 + (uo_ref[...] < 0).astype(jnp.int32)
    sd = ds_ref[...] * 4 + 2 + (do_ref[...] < 0).astype(jnp.int32)
    src_all = jnp.concatenate([su, sd, jnp.zeros((_PADR, 128), jnp.int32)],
                              axis=0)
    pad_dst = _N + lax.broadcasted_iota(jnp.int32, (_PADR, 128), 1)
    dst_all = jnp.concatenate([ud_ref[...], dd_ref[...], pad_dst], axis=0)
    idx2_ref[...] = jnp.stack(
        [src_all[:, 0:64], dst_all[:, 0:64],
         src_all[:, 64:128], dst_all[:, 64:128]], axis=1)


def _build_indices(up_src, up_orient, up_dst, down_src, down_orient, down_dst):
    out = pl.pallas_call(
        _idx_body,
        out_shape=jax.ShapeDtypeStruct((_EP // 128, 4, 64), jnp.int32),
    )(up_src.reshape(_ER, 128), up_orient.reshape(_ER, 128),
      up_dst.reshape(_ER, 128), down_src.reshape(_ER, 128),
      down_orient.reshape(_ER, 128), down_dst.reshape(_ER, 128))
    return out.reshape(_EP // 32, 64)


_RB = 2000


def _mm_body1(x_ref, wcat_ref, wt_ref, t_ref, hw_ref):
    xs = x_ref[...]
    t_ref[...] = jnp.dot(xs, wcat_ref[...], preferred_element_type=jnp.float32)
    hw_ref[...] = jnp.dot(xs, wt_ref[...], preferred_element_type=jnp.float32)


def _mm_body3(a0_ref, a1_ref, hwp_ref, wcat_ref, wt_ref, t_ref, hw_ref):
    xs = a0_ref[...] + a1_ref[...] + hwp_ref[...]
    t_ref[...] = jnp.dot(xs, wcat_ref[...], preferred_element_type=jnp.float32)
    hw_ref[...] = jnp.dot(xs, wt_ref[...], preferred_element_type=jnp.float32)


def _layer_matmul(terms, wcat_t, w_t):
    body = _mm_body1 if len(terms) == 1 else _mm_body3
    row_spec = pl.BlockSpec((_RB, 128), lambda i: (i, 0))
    in_specs = [row_spec] * len(terms) + [
        pl.BlockSpec((128, 512), lambda i: (0, 0)),
        pl.BlockSpec((128, 128), lambda i: (0, 0)),
    ]
    t_raw, hw = pl.pallas_call(
        body,
        grid=(_N // _RB,),
        in_specs=in_specs,
        out_specs=(
            pl.BlockSpec((_RB, 512), lambda i: (i, 0)),
            pl.BlockSpec((_RB, 128), lambda i: (i, 0)),
        ),
        out_shape=(
            jax.ShapeDtypeStruct((_N, 512), jnp.float32),
            jax.ShapeDtypeStruct((_N, 128), jnp.float32),
        ),
    )(*terms, wcat_t, w_t)
    return t_raw.reshape(4 * _N, 128), hw


_SC_MESH = plsc.VectorSubcoreMesh(core_axis_name="c", subcore_axis_name="s")


@functools.partial(
    pl.kernel,
    out_type=jax.ShapeDtypeStruct((_NC, _N, 128), jnp.float32),
    mesh=_SC_MESH,
    scratch_types=(
        [pltpu.VMEM((8, 64), jnp.int32)] * 2 +
        [pltpu.VMEM((_K, 128), jnp.float32)] * 4 +
        [pltpu.VMEM_SHARED((_N + 128, 128), jnp.float32)] +
        [pltpu.SemaphoreType.DMA] * 2 +
        [pltpu.SemaphoreType.DMA] * 4 +
        [pltpu.SemaphoreType.DMA] * 4
    ),
)
def _sc_scatter(table_hbm, idx2_hbm, zeros_hbm, out_hbm, *scr):
    ibuf = scr[0:2]
    rows = (scr[2:4], scr[4:6])
    acc = scr[6]
    isem = scr[7:9]
    gsem = (scr[9:11], scr[11:13])
    ssem = (scr[13:15], scr[15:17])
    c = lax.axis_index("c")
    s = lax.axis_index("s")
    chunk_base = lax.select(c == 0, s * _C0_CHUNKS,
                            _NS * _C0_CHUNKS + s * _C1_CHUNKS)
    my_nr = lax.select(c == 0, _NR0, _NR1)
    ibase = 2 * chunk_base
    r0 = s * _ROWS_PER_TILE
    tail = _NS * _ROWS_PER_TILE
    pltpu.sync_copy(zeros_hbm.at[pl.ds(r0, _ROWS_PER_TILE)],
                    acc.at[pl.ds(r0, _ROWS_PER_TILE)])

    @pl.when(s == _NS - 1)
    def _():
        pltpu.sync_copy(zeros_hbm.at[pl.ds(tail, _N - tail)],
                        acc.at[pl.ds(tail, _N - tail)])

    plsc.subcore_barrier()

    def ifetch(p, m):
        pltpu.async_copy(idx2_hbm.at[pl.ds(ibase + 8 * p, 8)],
                         ibuf[m], isem[m])

    def iwait(p, m):
        pltpu.make_async_copy(idx2_hbm.at[pl.ds(ibase + 8 * p, 8)],
                              ibuf[m], isem[m]).wait()

    def start_gathers(m, h, g):
        for b in range(2):
            pltpu.async_copy(table_hbm.at[ibuf[m].at[4 * h + 2 * b]],
                             rows[g][b], gsem[g][b])

    def wait_gather(m, h, g, b):
        pltpu.make_async_copy(table_hbm.at[ibuf[m].at[4 * h + 2 * b]],
                              rows[g][b], gsem[g][b]).wait()

    def start_scatter(m, h, g, b):
        pltpu.async_copy(rows[g][b], acc.at[ibuf[m].at[4 * h + 2 * b + 1]],
                         ssem[g][b], add=True)

    def wait_scatter(m, h, g, b):
        pltpu.make_async_copy(rows[g][b], acc.at[ibuf[m].at[4 * h + 2 * b + 1]],
                              ssem[g][b]).wait()

    ifetch(0, 0)
    iwait(0, 0)
    start_gathers(0, 0, 0)

    def round_t(t, u):
        g = u % 2
        m, h = (u // 2) % 2, u % 2
        un = (u + 1) % 4
        mn, hn = (un // 2) % 2, un % 2
        up = (u + 3) % 4
        mp, hp = (up // 2) % 2, up % 2
        for b in range(2):
            wait_gather(m, h, g, b)
            start_scatter(m, h, g, b)

        @pl.when(t + 1 < my_nr)
        def _():
            @pl.when(t >= 1)
            def _():
                for b in range(2):
                    wait_scatter(mp, hp, 1 - g, b)
            if un % 2 == 0:
                iwait((t + 1) // 2, mn)
            start_gathers(mn, hn, 1 - g)

        if u % 2 == 0:
            @pl.when(t + 2 < my_nr)
            def _():
                ifetch((t + 2) // 2, ((u + 2) // 2) % 2)

    def quad_body(qq, carry):
        for u in range(4):
            round_t(4 * qq + u, u)
        return carry

    lax.fori_loop(0, my_nr // 4, quad_body, 0)
    for b in range(2):
        wait_scatter(1, 0, 0, b)
    for b in range(2):
        wait_scatter(1, 1, 1, b)
    plsc.subcore_barrier()
    pltpu.sync_copy(acc.at[pl.ds(r0, _ROWS_PER_TILE)],
                    out_hbm.at[c, pl.ds(r0, _ROWS_PER_TILE)])

    @pl.when(s == _NS - 1)
    def _():
        pltpu.sync_copy(acc.at[pl.ds(tail, _N - tail)],
                        out_hbm.at[c, pl.ds(tail, _N - tail)])


def _pool_body(a0_ref, a1_ref, hw_ref, bt_ref, w1_ref, b1_ref, w2_ref, b2_ref,
               out_ref, pooled_ref):
    i = pl.program_id(0)
    h = jnp.abs(a0_ref[...] + a1_ref[...] + hw_ref[...])
    onehot = (bt_ref[...] == lax.broadcasted_iota(jnp.int32, (1, _B), 1)
              ).astype(jnp.float32)
    part = lax.dot_general(onehot, h, (((0,), (0,)), ((), ())),
                           preferred_element_type=jnp.float32)

    @pl.when(i == 0)
    def _():
        pooled_ref[...] = part

    @pl.when(i > 0)
    def _():
        pooled_ref[...] += part

    @pl.when(i == _N // _RB - 1)
    def _():
        p = pooled_ref[...]
        h1 = jnp.maximum(
            jnp.dot(p, w1_ref[...], preferred_element_type=jnp.float32)
            + b1_ref[...], 0.0)
        out_ref[...] = jnp.dot(h1, w2_ref[...],
                               preferred_element_type=jnp.float32) + b2_ref[...]


def _pool_mlp(a0, a1, hw, batch2d, w1t, b1, w2t, b2):
    row_spec = pl.BlockSpec((_RB, 128), lambda i: (i, 0))
    const = lambda shape: pl.BlockSpec(shape, lambda i: (0, 0))
    return pl.pallas_call(
        _pool_body,
        grid=(_N // _RB,),
        in_specs=[row_spec, row_spec, row_spec,
                  pl.BlockSpec((_RB, 1), lambda i: (i, 0)),
                  const((128, 128)), const((1, 128)),
                  const((128, 128)), const((1, 128))],
        out_specs=const((_B, 128)),
        out_shape=jax.ShapeDtypeStruct((_B, 128), jnp.float32),
        scratch_shapes=[pltpu.VMEM((_B, 128), jnp.float32)],
    )(a0, a1, hw, batch2d, w1t, b1, w2t, b2)


def kernel(x, up_index, up_orient, down_index, down_orient, batch,
           W_up_0, W_down_0, W_0, W_up_1, W_down_1, W_1, W_up_2, W_down_2, W_2,
           lin1_W, lin1_b, lin2_W, lin2_b):
    f32 = jnp.float32
    idx2 = _build_indices(up_index[0], up_orient, up_index[1],
                          down_index[0], down_orient, down_index[1])
    zeros = jnp.zeros((_N, 128), f32)

    def wcat(Wu, Wd):
        return jnp.concatenate([Wu.T, -Wu.T, Wd.T, -Wd.T], axis=1)

    layers = ((W_up_0, W_down_0, W_0), (W_up_1, W_down_1, W_1),
              (W_up_2, W_down_2, W_2))

    terms = (x,)
    for Wu, Wd, W in layers:
        t_tab, hw = _layer_matmul(terms, wcat(Wu, Wd), W.T)
        acc = _sc_scatter(t_tab, idx2, zeros)
        terms = (acc[0], acc[1], hw)

    w2p = jnp.zeros((128, 128), f32).at[:, :2].set(lin2_W.T)
    b2p = jnp.zeros((1, 128), f32).at[0, :2].set(lin2_b)
    out = _pool_mlp(terms[0], terms[1], terms[2], batch.reshape(_N, 1),
                    lin1_W.T, lin1_b.reshape(1, 128), w2p, b2p)
    return out[:, :2]


# speedup vs baseline: 1.1640x; 1.0838x over previous
import functools
import jax
import jax.numpy as jnp
from jax import lax
from jax.experimental import pallas as pl
from jax.experimental.pallas import tpu as pltpu
from jax.experimental.pallas import tpu_sc as plsc

_N = 10000
_D = 128
_H = 128
_E = 320000
_B = 8

_NC = 2
_NS = 16
_NW = _NC * _NS
_K = 64
_EPW = 20480
_EP = _NW * _EPW
_ER = _E // 128
_PADR = _EP // 128 - 2 * _ER
_CPT = _EPW // _K
_C0_CHUNKS = 456
_C1_CHUNKS = 2 * _CPT - _C0_CHUNKS
_NR0 = _C0_CHUNKS // 2
_NR1 = _C1_CHUNKS // 2
_ROWS_PER_TILE = 624


def _idx_body(us_ref, uo_ref, ud_ref, ds_ref, do_ref, dd_ref, idx2_ref):
    su = us_ref[...] * 4 + (uo_ref[...] < 0).astype(jnp.int32)
    sd = ds_ref[...] * 4 + 2 + (do_ref[...] < 0).astype(jnp.int32)
    src_all = jnp.concatenate([su, sd, jnp.zeros((_PADR, 128), jnp.int32)],
                              axis=0)
    pad_dst = _N + lax.broadcasted_iota(jnp.int32, (_PADR, 128), 1)
    dst_all = jnp.concatenate([ud_ref[...], dd_ref[...], pad_dst], axis=0)
    idx2_ref[...] = jnp.stack(
        [src_all[:, 0:64], dst_all[:, 0:64],
         src_all[:, 64:128], dst_all[:, 64:128]], axis=1)


def _build_indices(up_src, up_orient, up_dst, down_src, down_orient, down_dst):
    out = pl.pallas_call(
        _idx_body,
        out_shape=jax.ShapeDtypeStruct((_EP // 128, 4, 64), jnp.int32),
    )(up_src.reshape(_ER, 128), up_orient.reshape(_ER, 128),
      up_dst.reshape(_ER, 128), down_src.reshape(_ER, 128),
      down_orient.reshape(_ER, 128), down_dst.reshape(_ER, 128))
    return out.reshape(_EP // 32, 64)


_RB = 2000


def _mm_body1(x_ref, wcat_ref, wt_ref, t_ref, hw_ref):
    xs = x_ref[...]
    t_ref[...] = jnp.dot(xs, wcat_ref[...], preferred_element_type=jnp.float32)
    hw_ref[...] = jnp.dot(xs, wt_ref[...], preferred_element_type=jnp.float32)


def _mm_body3(a0_ref, a1_ref, hwp_ref, wcat_ref, wt_ref, t_ref, hw_ref):
    xs = a0_ref[...] + a1_ref[...] + hwp_ref[...]
    t_ref[...] = jnp.dot(xs, wcat_ref[...], preferred_element_type=jnp.float32)
    hw_ref[...] = jnp.dot(xs, wt_ref[...], preferred_element_type=jnp.float32)


def _layer_matmul(terms, wcat_t, w_t):
    body = _mm_body1 if len(terms) == 1 else _mm_body3
    row_spec = pl.BlockSpec((_RB, 128), lambda i: (i, 0))
    in_specs = [row_spec] * len(terms) + [
        pl.BlockSpec((128, 512), lambda i: (0, 0)),
        pl.BlockSpec((128, 128), lambda i: (0, 0)),
    ]
    t_raw, hw = pl.pallas_call(
        body,
        grid=(_N // _RB,),
        in_specs=in_specs,
        out_specs=(
            pl.BlockSpec((_RB, 512), lambda i: (i, 0)),
            pl.BlockSpec((_RB, 128), lambda i: (i, 0)),
        ),
        out_shape=(
            jax.ShapeDtypeStruct((_N, 512), jnp.float32),
            jax.ShapeDtypeStruct((_N, 128), jnp.float32),
        ),
    )(*terms, wcat_t, w_t)
    return t_raw.reshape(4 * _N, 128), hw


_SC_MESH = plsc.VectorSubcoreMesh(core_axis_name="c", subcore_axis_name="s")


@functools.partial(
    pl.kernel,
    out_type=jax.ShapeDtypeStruct((_NC, _N, 128), jnp.float32),
    mesh=_SC_MESH,
    scratch_types=(
        [pltpu.VMEM((8, 64), jnp.int32)] * 2 +
        [pltpu.VMEM((_K, 128), jnp.float32)] * 4 +
        [pltpu.VMEM_SHARED((_N + 128, 128), jnp.float32)] +
        [pltpu.SemaphoreType.DMA] * 2 +
        [pltpu.SemaphoreType.DMA] * 4 +
        [pltpu.SemaphoreType.DMA] * 4
    ),
)
def _sc_scatter(table_hbm, idx2_hbm, zeros_hbm, out_hbm, *scr):
    ibuf = scr[0:2]
    rows = (scr[2:4], scr[4:6])
    acc = scr[6]
    isem = scr[7:9]
    gsem = (scr[9:11], scr[11:13])
    ssem = (scr[13:15], scr[15:17])
    c = lax.axis_index("c")
    s = lax.axis_index("s")
    chunk_base = lax.select(c == 0, s * _C0_CHUNKS,
                            _NS * _C0_CHUNKS + s * _C1_CHUNKS)
    my_nr = lax.select(c == 0, _NR0, _NR1)
    ibase = 2 * chunk_base
    r0 = s * _ROWS_PER_TILE
    tail = _NS * _ROWS_PER_TILE
    pltpu.sync_copy(zeros_hbm.at[pl.ds(r0, _ROWS_PER_TILE)],
                    acc.at[pl.ds(r0, _ROWS_PER_TILE)])

    @pl.when(s == _NS - 1)
    def _():
        pltpu.sync_copy(zeros_hbm.at[pl.ds(tail, _N - tail)],
                        acc.at[pl.ds(tail, _N - tail)])

    plsc.subcore_barrier()

    def ifetch(p, m):
        pltpu.async_copy(idx2_hbm.at[pl.ds(ibase + 8 * p, 8)],
                         ibuf[m], isem[m])

    def iwait(p, m):
        pltpu.make_async_copy(idx2_hbm.at[pl.ds(ibase + 8 * p, 8)],
                              ibuf[m], isem[m]).wait()

    def start_gathers(m, h, g):
        for b in range(2):
            pltpu.async_copy(table_hbm.at[ibuf[m].at[4 * h + 2 * b]],
                             rows[g][b], gsem[g][b])

    def wait_gather(m, h, g, b):
        pltpu.make_async_copy(table_hbm.at[ibuf[m].at[4 * h + 2 * b]],
                              rows[g][b], gsem[g][b]).wait()

    def start_scatter(m, h, g, b):
        pltpu.async_copy(rows[g][b], acc.at[ibuf[m].at[4 * h + 2 * b + 1]],
                         ssem[g][b], add=True)

    def wait_scatter(m, h, g, b):
        pltpu.make_async_copy(rows[g][b], acc.at[ibuf[m].at[4 * h + 2 * b + 1]],
                              ssem[g][b]).wait()

    ifetch(0, 0)
    iwait(0, 0)
    start_gathers(0, 0, 0)

    def round_t(t, u):
        g = u % 2
        m, h = (u // 2) % 2, u % 2
        un = (u + 1) % 4
        mn, hn = (un // 2) % 2, un % 2
        up = (u + 3) % 4
        mp, hp = (up // 2) % 2, up % 2
        for b in range(2):
            wait_gather(m, h, g, b)
            start_scatter(m, h, g, b)

        @pl.when(t + 1 < my_nr)
        def _():
            @pl.when(t >= 1)
            def _():
                for b in range(2):
                    wait_scatter(mp, hp, 1 - g, b)
            if un % 2 == 0:
                iwait((t + 1) // 2, mn)
            start_gathers(mn, hn, 1 - g)

        if u % 2 == 0:
            @pl.when(t + 2 < my_nr)
            def _():
                ifetch((t + 2) // 2, ((u + 2) // 2) % 2)

    def quad_body(qq, carry):
        for u in range(4):
            round_t(4 * qq + u, u)
        return carry

    lax.fori_loop(0, my_nr // 4, quad_body, 0)
    for b in range(2):
        wait_scatter(1, 0, 0, b)
    for b in range(2):
        wait_scatter(1, 1, 1, b)
    plsc.subcore_barrier()
    pltpu.sync_copy(acc.at[pl.ds(r0, _ROWS_PER_TILE)],
                    out_hbm.at[c, pl.ds(r0, _ROWS_PER_TILE)])

    @pl.when(s == _NS - 1)
    def _():
        pltpu.sync_copy(acc.at[pl.ds(tail, _N - tail)],
                        out_hbm.at[c, pl.ds(tail, _N - tail)])


def _pool_body(a0_ref, a1_ref, hw_ref, bt_ref, w1_ref, b1_ref, w2_ref, b2_ref,
               out_ref, pooled_ref):
    i = pl.program_id(0)
    h = jnp.abs(a0_ref[...] + a1_ref[...] + hw_ref[...])
    onehot = (bt_ref[...] == lax.broadcasted_iota(jnp.int32, (1, _B), 1)
              ).astype(jnp.float32)
    part = lax.dot_general(onehot, h, (((0,), (0,)), ((), ())),
                           preferred_element_type=jnp.float32)

    @pl.when(i == 0)
    def _():
        pooled_ref[...] = part

    @pl.when(i > 0)
    def _():
        pooled_ref[...] += part

    @pl.when(i == _N // _RB - 1)
    def _():
        p = pooled_ref[...]
        h1 = jnp.maximum(
            jnp.dot(p, w1_ref[...], preferred_element_type=jnp.float32)
            + b1_ref[...], 0.0)
        out_ref[...] = jnp.dot(h1, w2_ref[...],
                               preferred_element_type=jnp.float32) + b2_ref[...]


def _pool_mlp(a0, a1, hw, batch2d, w1t, b1, w2t, b2):
    row_spec = pl.BlockSpec((_RB, 128), lambda i: (i, 0))
    const = lambda shape: pl.BlockSpec(shape, lambda i: (0, 0))
    return pl.pallas_call(
        _pool_body,
        grid=(_N // _RB,),
        in_specs=[row_spec, row_spec, row_spec,
                  pl.BlockSpec((_RB, 1), lambda i: (i, 0)),
                  const((128, 128)), const((1, 128)),
                  const((128, 128)), const((1, 128))],
        out_specs=const((_B, 128)),
        out_shape=jax.ShapeDtypeStruct((_B, 128), jnp.float32),
        scratch_shapes=[pltpu.VMEM((_B, 128), jnp.float32)],
    )(a0, a1, hw, batch2d, w1t, b1, w2t, b2)


def kernel(x, up_index, up_orient, down_index, down_orient, batch,
           W_up_0, W_down_0, W_0, W_up_1, W_down_1, W_1, W_up_2, W_down_2, W_2,
           lin1_W, lin1_b, lin2_W, lin2_b):
    f32 = jnp.float32
    idx2 = _build_indices(up_index[0], up_orient, up_index[1],
                          down_index[0], down_orient, down_index[1])
    zeros = jnp.zeros((_N, 128), f32)

    def wcat(Wu, Wd):
        return jnp.concatenate([Wu.T, -Wu.T, Wd.T, -Wd.T], axis=1)

    layers = ((W_up_0, W_down_0, W_0), (W_up_1, W_down_1, W_1),
              (W_up_2, W_down_2, W_2))

    terms = (x,)
    for Wu, Wd, W in layers:
        t_tab, hw = _layer_matmul(terms, wcat(Wu, Wd), W.T)
        acc = _sc_scatter(t_tab, idx2, zeros)
        terms = (acc[0], acc[1], hw)

    w2p = jnp.zeros((128, 128), f32).at[:, :2].set(lin2_W.T)
    b2p = jnp.zeros((1, 128), f32).at[0, :2].set(lin2_b)
    out = _pool_mlp(terms[0], terms[1], terms[2], batch.reshape(_N, 1),
                    lin1_W.T, lin1_b.reshape(1, 128), w2p, b2p)
    return out[:, :2]


# speedup vs baseline: 1.1995x; 1.0305x over previous
import functools
import jax
import jax.numpy as jnp
from jax import lax
from jax.experimental import pallas as pl
from jax.experimental.pallas import tpu as pltpu
from jax.experimental.pallas import tpu_sc as plsc

_N = 10000
_D = 128
_H = 128
_E = 320000
_B = 8

_NC = 2
_NS = 16
_NW = _NC * _NS
_K = 64
_EPW = 20480
_EP = _NW * _EPW
_ER = _E // 128
_PADR = _EP // 128 - 2 * _ER
_CPT = _EPW // _K
_C0_CHUNKS = 512
_C1_CHUNKS = 2 * _CPT - _C0_CHUNKS
_NR0 = _C0_CHUNKS // 2
_NR1 = _C1_CHUNKS // 2
_ROWS_PER_TILE = 624


def _idx_body(us_ref, uo_ref, ud_ref, ds_ref, do_ref, dd_ref, idx2_ref):
    su = us_ref[...] * 4 + (uo_ref[...] < 0).astype(jnp.int32)
    sd = ds_ref[...] * 4 + 2 + (do_ref[...] < 0).astype(jnp.int32)
    src_all = jnp.concatenate([su, sd, jnp.zeros((_PADR, 128), jnp.int32)],
                              axis=0)
    pad_dst = _N + lax.broadcasted_iota(jnp.int32, (_PADR, 128), 1)
    dst_all = jnp.concatenate([ud_ref[...], dd_ref[...], pad_dst], axis=0)
    idx2_ref[...] = jnp.stack(
        [src_all[:, 0:64], dst_all[:, 0:64],
         src_all[:, 64:128], dst_all[:, 64:128]], axis=1)


def _build_indices(up_src, up_orient, up_dst, down_src, down_orient, down_dst):
    out = pl.pallas_call(
        _idx_body,
        out_shape=jax.ShapeDtypeStruct((_EP // 128, 4, 64), jnp.int32),
    )(up_src.reshape(_ER, 128), up_orient.reshape(_ER, 128),
      up_dst.reshape(_ER, 128), down_src.reshape(_ER, 128),
      down_orient.reshape(_ER, 128), down_dst.reshape(_ER, 128))
    return out.reshape(_EP // 32, 64)


_RB = 2000


def _mm_body1(x_ref, wcat_ref, wt_ref, t_ref, hw_ref):
    xs = x_ref[...]
    t_ref[...] = jnp.dot(xs, wcat_ref[...], preferred_element_type=jnp.float32)
    hw_ref[...] = jnp.dot(xs, wt_ref[...], preferred_element_type=jnp.float32)


def _mm_body3(a0_ref, a1_ref, hwp_ref, wcat_ref, wt_ref, t_ref, hw_ref):
    xs = a0_ref[...] + a1_ref[...] + hwp_ref[...]
    t_ref[...] = jnp.dot(xs, wcat_ref[...], preferred_element_type=jnp.float32)
    hw_ref[...] = jnp.dot(xs, wt_ref[...], preferred_element_type=jnp.float32)


def _layer_matmul(terms, wcat_t, w_t):
    body = _mm_body1 if len(terms) == 1 else _mm_body3
    row_spec = pl.BlockSpec((_RB, 128), lambda i: (i, 0))
    in_specs = [row_spec] * len(terms) + [
        pl.BlockSpec((128, 512), lambda i: (0, 0)),
        pl.BlockSpec((128, 128), lambda i: (0, 0)),
    ]
    t_raw, hw = pl.pallas_call(
        body,
        grid=(_N // _RB,),
        in_specs=in_specs,
        out_specs=(
            pl.BlockSpec((_RB, 512), lambda i: (i, 0)),
            pl.BlockSpec((_RB, 128), lambda i: (i, 0)),
        ),
        out_shape=(
            jax.ShapeDtypeStruct((_N, 512), jnp.float32),
            jax.ShapeDtypeStruct((_N, 128), jnp.float32),
        ),
    )(*terms, wcat_t, w_t)
    return t_raw.reshape(4 * _N, 128), hw


_SC_MESH = plsc.VectorSubcoreMesh(core_axis_name="c", subcore_axis_name="s")


@functools.partial(
    pl.kernel,
    out_type=jax.ShapeDtypeStruct((_NC, _N, 128), jnp.float32),
    mesh=_SC_MESH,
    scratch_types=(
        [pltpu.VMEM((8, 64), jnp.int32)] * 2 +
        [pltpu.VMEM((_K, 128), jnp.float32)] * 4 +
        [pltpu.VMEM_SHARED((_N + 128, 128), jnp.float32)] +
        [pltpu.SemaphoreType.DMA] * 2 +
        [pltpu.SemaphoreType.DMA] * 4 +
        [pltpu.SemaphoreType.DMA] * 4
    ),
)
def _sc_scatter(table_hbm, idx2_hbm, zeros_hbm, out_hbm, *scr):
    ibuf = scr[0:2]
    rows = (scr[2:4], scr[4:6])
    acc = scr[6]
    isem = scr[7:9]
    gsem = (scr[9:11], scr[11:13])
    ssem = (scr[13:15], scr[15:17])
    c = lax.axis_index("c")
    s = lax.axis_index("s")
    chunk_base = lax.select(c == 0, s * _C0_CHUNKS,
                            _NS * _C0_CHUNKS + s * _C1_CHUNKS)
    my_nr = lax.select(c == 0, _NR0, _NR1)
    ibase = 2 * chunk_base
    r0 = s * _ROWS_PER_TILE
    tail = _NS * _ROWS_PER_TILE
    pltpu.sync_copy(zeros_hbm.at[pl.ds(r0, _ROWS_PER_TILE)],
                    acc.at[pl.ds(r0, _ROWS_PER_TILE)])

    @pl.when(s == _NS - 1)
    def _():
        pltpu.sync_copy(zeros_hbm.at[pl.ds(tail, _N - tail)],
                        acc.at[pl.ds(tail, _N - tail)])

    plsc.subcore_barrier()

    def ifetch(p, m):
        pltpu.async_copy(idx2_hbm.at[pl.ds(ibase + 8 * p, 8)],
                         ibuf[m], isem[m])

    def iwait(p, m):
        pltpu.make_async_copy(idx2_hbm.at[pl.ds(ibase + 8 * p, 8)],
                              ibuf[m], isem[m]).wait()

    def start_gathers(m, h, g):
        for b in range(2):
            pltpu.async_copy(table_hbm.at[ibuf[m].at[4 * h + 2 * b]],
                             rows[g][b], gsem[g][b])

    def wait_gather(m, h, g, b):
        pltpu.make_async_copy(table_hbm.at[ibuf[m].at[4 * h + 2 * b]],
                              rows[g][b], gsem[g][b]).wait()

    def start_scatter(m, h, g, b):
        pltpu.async_copy(rows[g][b], acc.at[ibuf[m].at[4 * h + 2 * b + 1]],
                         ssem[g][b], add=True)

    def wait_scatter(m, h, g, b):
        pltpu.make_async_copy(rows[g][b], acc.at[ibuf[m].at[4 * h + 2 * b + 1]],
                              ssem[g][b]).wait()

    ifetch(0, 0)
    iwait(0, 0)
    start_gathers(0, 0, 0)

    def round_t(t, u):
        g = u % 2
        m, h = (u // 2) % 2, u % 2
        un = (u + 1) % 4
        mn, hn = (un // 2) % 2, un % 2
        up = (u + 3) % 4
        mp, hp = (up // 2) % 2, up % 2
        for b in range(2):
            wait_gather(m, h, g, b)
            start_scatter(m, h, g, b)

        @pl.when(t + 1 < my_nr)
        def _():
            @pl.when(t >= 1)
            def _():
                for b in range(2):
                    wait_scatter(mp, hp, 1 - g, b)
            if un % 2 == 0:
                iwait((t + 1) // 2, mn)
            start_gathers(mn, hn, 1 - g)

        if u % 2 == 0:
            @pl.when(t + 2 < my_nr)
            def _():
                ifetch((t + 2) // 2, ((u + 2) // 2) % 2)

    def quad_body(qq, carry):
        for u in range(4):
            round_t(4 * qq + u, u)
        return carry

    lax.fori_loop(0, my_nr // 4, quad_body, 0)
    for b in range(2):
        wait_scatter(1, 0, 0, b)
    for b in range(2):
        wait_scatter(1, 1, 1, b)
    plsc.subcore_barrier()
    pltpu.sync_copy(acc.at[pl.ds(r0, _ROWS_PER_TILE)],
                    out_hbm.at[c, pl.ds(r0, _ROWS_PER_TILE)])

    @pl.when(s == _NS - 1)
    def _():
        pltpu.sync_copy(acc.at[pl.ds(tail, _N - tail)],
                        out_hbm.at[c, pl.ds(tail, _N - tail)])


def _pool_body(a0_ref, a1_ref, hw_ref, bt_ref, w1_ref, b1_ref, w2_ref, b2_ref,
               out_ref, pooled_ref):
    i = pl.program_id(0)
    h = jnp.abs(a0_ref[...] + a1_ref[...] + hw_ref[...])
    onehot = (bt_ref[...] == lax.broadcasted_iota(jnp.int32, (1, _B), 1)
              ).astype(jnp.float32)
    part = lax.dot_general(onehot, h, (((0,), (0,)), ((), ())),
                           preferred_element_type=jnp.float32)

    @pl.when(i == 0)
    def _():
        pooled_ref[...] = part

    @pl.when(i > 0)
    def _():
        pooled_ref[...] += part

    @pl.when(i == _N // _RB - 1)
    def _():
        p = pooled_ref[...]
        h1 = jnp.maximum(
            jnp.dot(p, w1_ref[...], preferred_element_type=jnp.float32)
            + b1_ref[...], 0.0)
        out_ref[...] = jnp.dot(h1, w2_ref[...],
                               preferred_element_type=jnp.float32) + b2_ref[...]


def _pool_mlp(a0, a1, hw, batch2d, w1t, b1, w2t, b2):
    row_spec = pl.BlockSpec((_RB, 128), lambda i: (i, 0))
    const = lambda shape: pl.BlockSpec(shape, lambda i: (0, 0))
    return pl.pallas_call(
        _pool_body,
        grid=(_N // _RB,),
        in_specs=[row_spec, row_spec, row_spec,
                  pl.BlockSpec((_RB, 1), lambda i: (i, 0)),
                  const((128, 128)), const((1, 128)),
                  const((128, 128)), const((1, 128))],
        out_specs=const((_B, 128)),
        out_shape=jax.ShapeDtypeStruct((_B, 128), jnp.float32),
        scratch_shapes=[pltpu.VMEM((_B, 128), jnp.float32)],
    )(a0, a1, hw, batch2d, w1t, b1, w2t, b2)


def kernel(x, up_index, up_orient, down_index, down_orient, batch,
           W_up_0, W_down_0, W_0, W_up_1, W_down_1, W_1, W_up_2, W_down_2, W_2,
           lin1_W, lin1_b, lin2_W, lin2_b):
    f32 = jnp.float32
    idx2 = _build_indices(up_index[0], up_orient, up_index[1],
                          down_index[0], down_orient, down_index[1])
    zeros = jnp.zeros((_N, 128), f32)

    def wcat(Wu, Wd):
        return jnp.concatenate([Wu.T, -Wu.T, Wd.T, -Wd.T], axis=1)

    layers = ((W_up_0, W_down_0, W_0), (W_up_1, W_down_1, W_1),
              (W_up_2, W_down_2, W_2))

    terms = (x,)
    for Wu, Wd, W in layers:
        t_tab, hw = _layer_matmul(terms, wcat(Wu, Wd), W.T)
        acc = _sc_scatter(t_tab, idx2, zeros)
        terms = (acc[0], acc[1], hw)

    w2p = jnp.zeros((128, 128), f32).at[:, :2].set(lin2_W.T)
    b2p = jnp.zeros((1, 128), f32).at[0, :2].set(lin2_b)
    out = _pool_mlp(terms[0], terms[1], terms[2], batch.reshape(_N, 1),
                    lin1_W.T, lin1_b.reshape(1, 128), w2p, b2p)
    return out[:, :2]
